# baseline (speedup 1.0000x reference)
import jax
import jax.numpy as jnp
from jax.experimental import pallas as pl
from jax.experimental.pallas import tpu as pltpu

_W = 15
_P = _W // 2
_H = 512
_HP = _H + 2 * _P
_RP = 544


def _pad_rows(x, n_cols):
    top = [x[k:k + 1, :] for k in range(_P, 0, -1)]
    bot = [x[k:k + 1, :] for k in range(_H - 2, _H - 2 - _P, -1)]
    inf = jnp.full((_RP - _HP, n_cols), jnp.inf, dtype=x.dtype)
    return jnp.concatenate(top + [x] + bot + [inf], axis=0)


def _slide_min_rows(x):
    rows, n_cols = x.shape
    t = x.reshape(rows // 8, 8, n_cols)
    iota = jax.lax.broadcasted_iota(jnp.int32, (1, 8, n_cols), 1)

    def step(u, k):
        ur = pltpu.roll(u, 8 - k, axis=1)
        nxt = jnp.concatenate([ur[1:], ur[:1]], axis=0)
        return jnp.minimum(u, jnp.where(iota < 8 - k, ur, nxt))

    a = step(t, 1)
    b = step(a, 2)
    c = step(b, 4)
    d = step(c, 7)
    return d.reshape(rows, n_cols)[:_HP]


def _slide_min8_rows(x):
    rows, n_cols = x.shape
    t = x.reshape(rows // 8, 8, n_cols)
    iota = jax.lax.broadcasted_iota(jnp.int32, (1, 8, n_cols), 1)

    def step(u, k):
        ur = pltpu.roll(u, 8 - k, axis=1)
        nxt = jnp.concatenate([ur[1:], ur[:1]], axis=0)
        return jnp.minimum(u, jnp.where(iota < 8 - k, ur, nxt))

    c = step(step(step(t, 1), 2), 4)
    return c.reshape(rows, n_cols)[:536]


def _dark_channel_sum(m):
    v = _slide_min_rows(_pad_rows(m, m.shape[1]))

    vt = v.T

    dc = _slide_min_rows(_pad_rows(vt, _HP))
    return jnp.sum(dc)


def _dark_channel_kernel(x_ref, out_ref):
    s = jnp.float32(0)
    for i in range(x_ref.shape[0]):
        m = jnp.minimum(
            jnp.minimum(x_ref[i, 0], x_ref[i, 1]), x_ref[i, 2])
        s = s + _dark_channel_sum(m)
    out_ref[0] = jnp.reshape(s, (1, 1))


def kernel(generated_image):
    B = generated_image.shape[0]
    partial = pl.pallas_call(
        _dark_channel_kernel,
        grid=(B // 8,),
        in_specs=[pl.BlockSpec((8, 3, _H, _H), lambda b: (b, 0, 0, 0))],
        out_specs=pl.BlockSpec((1, 1, 1), lambda b: (b, 0, 0)),
        out_shape=jax.ShapeDtypeStruct((B // 8, 1, 1), jnp.float32),
        compiler_params=pltpu.CompilerParams(
            dimension_semantics=("arbitrary",),
        ),
    )(generated_image)
    return -(jnp.sum(partial) / (B * _HP * _HP))


# speedup vs baseline: 1.0639x; 1.0639x over previous
import jax
import jax.numpy as jnp
from jax.experimental import pallas as pl
from jax.experimental.pallas import tpu as pltpu

_W = 15
_P = _W // 2
_H = 512
_HP = _H + 2 * _P
_RP = 544


def _pad_rows(x, n_cols):
    top = [x[k:k + 1, :] for k in range(_P, 0, -1)]
    bot = [x[k:k + 1, :] for k in range(_H - 2, _H - 2 - _P, -1)]
    inf = jnp.full((_RP - _HP, n_cols), jnp.inf, dtype=x.dtype)
    return jnp.concatenate(top + [x] + bot + [inf], axis=0)


def _slide_min_rows(x):
    rows, n_cols = x.shape
    t = x.reshape(rows // 8, 8, n_cols)
    iota = jax.lax.broadcasted_iota(jnp.int32, (1, 8, n_cols), 1)

    def step(u, k):
        ur = pltpu.roll(u, 8 - k, axis=1)
        nxt = jnp.concatenate([ur[1:], ur[:1]], axis=0)
        return jnp.minimum(u, jnp.where(iota < 8 - k, ur, nxt))

    a = step(t, 1)
    b = step(a, 2)
    c = step(b, 4)
    d = step(c, 7)
    return d.reshape(rows, n_cols)[:_HP]


def _slide_min8_rows(x):
    rows, n_cols = x.shape
    t = x.reshape(rows // 8, 8, n_cols)
    iota = jax.lax.broadcasted_iota(jnp.int32, (1, 8, n_cols), 1)

    def step(u, k):
        ur = pltpu.roll(u, 8 - k, axis=1)
        nxt = jnp.concatenate([ur[1:], ur[:1]], axis=0)
        return jnp.minimum(u, jnp.where(iota < 8 - k, ur, nxt))

    c = step(step(step(t, 1), 2), 4)
    return c.reshape(rows, n_cols)[:536]


def _dark_channel_sum(m):
    v = _slide_min_rows(_pad_rows(m, m.shape[1]))

    vt = v.T

    dc = _slide_min_rows(_pad_rows(vt, _HP))
    return jnp.sum(dc)


def _dark_channel_kernel(x_ref, out_ref):
    s = jnp.float32(0)
    for i in range(x_ref.shape[0]):
        m = jnp.minimum(
            jnp.minimum(x_ref[i, 0], x_ref[i, 1]), x_ref[i, 2])
        s = s + _dark_channel_sum(m)
    out_ref[0] = jnp.reshape(s, (1, 1))


def kernel(generated_image):
    B = generated_image.shape[0]
    partial = pl.pallas_call(
        _dark_channel_kernel,
        grid=(B // 4,),
        in_specs=[pl.BlockSpec((4, 3, _H, _H), lambda b: (b, 0, 0, 0))],
        out_specs=pl.BlockSpec((1, 1, 1), lambda b: (b, 0, 0)),
        out_shape=jax.ShapeDtypeStruct((B // 4, 1, 1), jnp.float32),
        compiler_params=pltpu.CompilerParams(
            dimension_semantics=("arbitrary",),
        ),
    )(generated_image)
    return -(jnp.sum(partial) / (B * _HP * _HP))


# speedup vs baseline: 1.1153x; 1.0483x over previous
import jax
import jax.numpy as jnp
from jax.experimental import pallas as pl
from jax.experimental.pallas import tpu as pltpu

_W = 15
_P = _W // 2
_H = 512
_HP = _H + 2 * _P
_RP = 544


def _pad_rows(x, n_cols):
    top = [x[k:k + 1, :] for k in range(_P, 0, -1)]
    bot = [x[k:k + 1, :] for k in range(_H - 2, _H - 2 - _P, -1)]
    inf = jnp.full((_RP - _HP, n_cols), jnp.inf, dtype=x.dtype)
    return jnp.concatenate(top + [x] + bot + [inf], axis=0)


def _slide_min_rows(x):
    rows, n_cols = x.shape
    t = x.reshape(rows // 8, 8, n_cols)
    iota = jax.lax.broadcasted_iota(jnp.int32, (1, 8, n_cols), 1)

    def step(u, k):
        ur = pltpu.roll(u, 8 - k, axis=1)
        nxt = jnp.concatenate([ur[1:], ur[:1]], axis=0)
        return jnp.minimum(u, jnp.where(iota < 8 - k, ur, nxt))

    a = step(t, 1)
    b = step(a, 2)
    c = step(b, 4)
    d = step(c, 7)
    return d.reshape(rows, n_cols)[:_HP]


def _slide_min8_rows(x):
    rows, n_cols = x.shape
    t = x.reshape(rows // 8, 8, n_cols)
    iota = jax.lax.broadcasted_iota(jnp.int32, (1, 8, n_cols), 1)

    def step(u, k):
        ur = pltpu.roll(u, 8 - k, axis=1)
        nxt = jnp.concatenate([ur[1:], ur[:1]], axis=0)
        return jnp.minimum(u, jnp.where(iota < 8 - k, ur, nxt))

    c = step(step(step(t, 1), 2), 4)
    return c.reshape(rows, n_cols)[:536]


def _dark_channel_kernel(x_ref, out_ref):
    n = x_ref.shape[0]

    m = jnp.concatenate(
        [jnp.minimum(jnp.minimum(x_ref[i, 0], x_ref[i, 1]), x_ref[i, 2])
         for i in range(n)], axis=1)

    v = _slide_min_rows(_pad_rows(m, n * _H))

    vt = v.T

    blocks = [vt[i * _H:(i + 1) * _H, :_H] for i in range(n)]
    strips = [vt[i * _H:(i + 1) * _H, _H:] for i in range(n)]
    wide = jnp.concatenate(blocks + strips, axis=1)

    dc = _slide_min_rows(_pad_rows(wide, n * _HP))

    out_ref[0] = jnp.reshape(jnp.sum(dc), (1, 1))


def kernel(generated_image):
    B = generated_image.shape[0]
    partial = pl.pallas_call(
        _dark_channel_kernel,
        grid=(B // 4,),
        in_specs=[pl.BlockSpec((4, 3, _H, _H), lambda b: (b, 0, 0, 0))],
        out_specs=pl.BlockSpec((1, 1, 1), lambda b: (b, 0, 0)),
        out_shape=jax.ShapeDtypeStruct((B // 4, 1, 1), jnp.float32),
        compiler_params=pltpu.CompilerParams(
            dimension_semantics=("arbitrary",),
        ),
    )(generated_image)
    return -(jnp.sum(partial) / (B * _HP * _HP))


# speedup vs baseline: 1.1165x; 1.0011x over previous
import jax
import jax.numpy as jnp
from jax.experimental import pallas as pl
from jax.experimental.pallas import tpu as pltpu

_W = 15
_P = _W // 2
_H = 512
_HP = _H + 2 * _P
_RP = 544


def _pad_rows(x, n_cols):
    top = [x[k:k + 1, :] for k in range(_P, 0, -1)]
    bot = [x[k:k + 1, :] for k in range(_H - 2, _H - 2 - _P, -1)]
    inf = jnp.full((_RP - _HP, n_cols), jnp.inf, dtype=x.dtype)
    return jnp.concatenate(top + [x] + bot + [inf], axis=0)


def _slide_min_rows(x):
    rows, n_cols = x.shape
    t = x.reshape(rows // 8, 8, n_cols)
    iota = jax.lax.broadcasted_iota(jnp.int32, (1, 8, n_cols), 1)

    def step(u, k):
        ur = pltpu.roll(u, 8 - k, axis=1)
        nxt = jnp.concatenate([ur[1:], ur[:1]], axis=0)
        return jnp.minimum(u, jnp.where(iota < 8 - k, ur, nxt))

    a = step(t, 1)
    b = step(a, 2)
    c = step(b, 4)
    d = step(c, 7)
    return d.reshape(rows, n_cols)[:_HP]


def _slide_min8_rows(x):
    rows, n_cols = x.shape
    t = x.reshape(rows // 8, 8, n_cols)
    iota = jax.lax.broadcasted_iota(jnp.int32, (1, 8, n_cols), 1)

    def step(u, k):
        ur = pltpu.roll(u, 8 - k, axis=1)
        nxt = jnp.concatenate([ur[1:], ur[:1]], axis=0)
        return jnp.minimum(u, jnp.where(iota < 8 - k, ur, nxt))

    c = step(step(step(t, 1), 2), 4)
    return c.reshape(rows, n_cols)[:536]


def _dark_channel_pair_sum(x_ref, idx):
    n = len(idx)

    m = jnp.concatenate(
        [jnp.minimum(jnp.minimum(x_ref[i, 0], x_ref[i, 1]), x_ref[i, 2])
         for i in idx], axis=1)

    v = _slide_min_rows(_pad_rows(m, n * _H))

    vt = v.T

    blocks = [vt[i * _H:(i + 1) * _H, :_H] for i in range(n)]
    strips = [vt[i * _H:(i + 1) * _H, _H:] for i in range(n)]
    wide = jnp.concatenate(blocks + strips, axis=1)

    dc = _slide_min_rows(_pad_rows(wide, n * _HP))
    return jnp.sum(dc)


def _dark_channel_kernel(x_ref, out_ref):
    n = x_ref.shape[0]
    s = _dark_channel_pair_sum(x_ref, list(range(n)))
    out_ref[0] = jnp.reshape(s, (1, 1))


def kernel(generated_image):
    B = generated_image.shape[0]
    partial = pl.pallas_call(
        _dark_channel_kernel,
        grid=(B // 4,),
        in_specs=[pl.BlockSpec((4, 3, _H, _H), lambda b: (b, 0, 0, 0))],
        out_specs=pl.BlockSpec((1, 1, 1), lambda b: (b, 0, 0)),
        out_shape=jax.ShapeDtypeStruct((B // 4, 1, 1), jnp.float32),
        compiler_params=pltpu.CompilerParams(
            dimension_semantics=("arbitrary",),
        ),
    )(generated_image)
    return -(jnp.sum(partial) / (B * _HP * _HP))
